# baseline (speedup 1.0000x reference)
import functools

import jax
import jax.numpy as jnp
from jax import lax
from jax.experimental import pallas as pl
from jax.experimental.pallas import tpu as pltpu
from jax.experimental.pallas import tpu_sc as plsc

_B, _N, _P = 4096, 2048, 50
_MARGIN = 1.0
_NC, _NS, _L = 2, 16, 16
_NW = _NC * _NS
_ROWS_PER_W = _B // _NW
_R = 16
_CHUNKS = _ROWS_PER_W // _R
_IDXBUF = _R * _P + _L


def _loss_kernel(scores_hbm, pos_hbm, neg_hbm, out_hbm,
                 sc0, sc1, pb0, pb1, nb0, nb1, outv, sem0, sem1):
    wid = lax.axis_index("s") * _NC + lax.axis_index("c")
    row0 = wid * _ROWS_PER_W

    sbufs = (sc0, sc1)
    pbufs = (pb0, pb1)
    nbufs = (nb0, nb1)
    sems = (sem0, sem1)

    def start_chunk(c, slot):
        base = row0 + c * _R
        h = [
            pltpu.async_copy(
                scores_hbm.at[pl.ds(base * _N, _R * _N)],
                sbufs[slot], sems[slot]),
            pltpu.async_copy(
                pos_hbm.at[pl.ds(base * _P, _R * _P)],
                pbufs[slot].at[pl.ds(0, _R * _P)], sems[slot]),
            pltpu.async_copy(
                neg_hbm.at[pl.ds(base * _P, _R * _P)],
                nbufs[slot].at[pl.ds(0, _R * _P)], sems[slot]),
        ]
        return h

    lane = lax.iota(jnp.int32, _L)
    tail_valid = lane < (_P - 3 * _L)
    zero16 = jnp.zeros((_L,), jnp.float32)

    def make_row_body(sbuf, pbuf, nbuf):
        def row_body(r, acc):
            ibase = r * _P
            off = r * _N
            for g in range(4):
                pi = pbuf[pl.ds(ibase + g * _L, _L)]
                ni = nbuf[pl.ds(ibase + g * _L, _L)]
                if g == 3:
                    pi = jnp.where(tail_valid, pi, 0)
                    ni = jnp.where(tail_valid, ni, 0)
                ps = plsc.load_gather(sbuf, [pi + off])
                ns = plsc.load_gather(sbuf, [ni + off])
                loss = jnp.maximum(_MARGIN + ns - ps, 0.0)
                if g == 3:
                    loss = jnp.where(tail_valid, loss, zero16)
                acc = acc + loss
            return acc
        return row_body

    pending = start_chunk(0, 0)
    acc = zero16
    for c in range(_CHUNKS):
        slot = c % 2
        for h in pending:
            h.wait()
        if c + 1 < _CHUNKS:
            pending = start_chunk(c + 1, (c + 1) % 2)
        acc = lax.fori_loop(
            0, _R, make_row_body(sbufs[slot], pbufs[slot], nbufs[slot]), acc)

    outv[...] = acc * (1.0 / (_B * _P))
    pltpu.sync_copy(outv, out_hbm.at[wid])


@jax.jit
def kernel(saliency_scores, pos_indices, neg_indices):
    mesh = plsc.VectorSubcoreMesh(core_axis_name="c", subcore_axis_name="s")
    run = functools.partial(
        pl.kernel,
        out_type=jax.ShapeDtypeStruct((_NW, _L), jnp.float32),
        mesh=mesh,
        compiler_params=pltpu.CompilerParams(needs_layout_passes=False),
        scratch_types=[
            pltpu.VMEM((_R * _N,), jnp.float32),
            pltpu.VMEM((_R * _N,), jnp.float32),
            pltpu.VMEM((_IDXBUF,), jnp.int32),
            pltpu.VMEM((_IDXBUF,), jnp.int32),
            pltpu.VMEM((_IDXBUF,), jnp.int32),
            pltpu.VMEM((_IDXBUF,), jnp.int32),
            pltpu.VMEM((_L,), jnp.float32),
            pltpu.SemaphoreType.DMA,
            pltpu.SemaphoreType.DMA,
        ],
    )(_loss_kernel)
    partials = run(
        saliency_scores.reshape(-1),
        pos_indices.astype(jnp.int32).reshape(-1),
        neg_indices.astype(jnp.int32).reshape(-1),
    )
    return jnp.sum(partials)


# speedup vs baseline: 1.5865x; 1.5865x over previous
import functools

import jax
import jax.numpy as jnp
from jax import lax
from jax.experimental import pallas as pl
from jax.experimental.pallas import tpu as pltpu
from jax.experimental.pallas import tpu_sc as plsc

_B, _N, _P = 4096, 2048, 50
_MARGIN = 1.0
_NC, _NS, _L = 2, 16, 16
_NW = _NC * _NS
_ROWS_PER_W = _B // _NW
_R = 16
_CHUNKS = _ROWS_PER_W // _R
_GROUPS = (0, 16, 32, 34)


def _loss_kernel(scores_hbm, pos_hbm, neg_hbm, out_hbm,
                 sc0, sc1, pb0, pb1, nb0, nb1, outv, sem0, sem1):
    wid = lax.axis_index("s") * _NC + lax.axis_index("c")
    row0 = wid * _ROWS_PER_W

    sbufs = (sc0, sc1)
    pbufs = (pb0, pb1)
    nbufs = (nb0, nb1)
    sems = (sem0, sem1)

    def start_chunk(c, slot):
        base = row0 + c * _R
        return [
            pltpu.async_copy(scores_hbm.at[pl.ds(base, _R)], sbufs[slot],
                             sems[slot]),
            pltpu.async_copy(pos_hbm.at[pl.ds(base, _R)], pbufs[slot],
                             sems[slot]),
            pltpu.async_copy(neg_hbm.at[pl.ds(base, _R)], nbufs[slot],
                             sems[slot]),
        ]

    lane = lax.iota(jnp.int32, _L)
    tail_new = lane >= (3 * _L - _GROUPS[3])
    zero16 = jnp.zeros((_L,), jnp.float32)

    def make_row_body(sbuf, pbuf, nbuf):
        def row_body(r, acc):
            rv = jnp.full((_L,), 0, jnp.int32) + r
            for g, start in enumerate(_GROUPS):
                pi = pbuf[r, pl.ds(start, _L)]
                ni = nbuf[r, pl.ds(start, _L)]
                ps = plsc.load_gather(sbuf, [rv, pi])
                ns = plsc.load_gather(sbuf, [rv, ni])
                loss = jnp.maximum(_MARGIN + ns - ps, 0.0)
                if g == 3:
                    loss = jnp.where(tail_new, loss, zero16)
                acc = acc + loss
            return acc
        return row_body

    pending = start_chunk(0, 0)
    acc = zero16
    for c in range(_CHUNKS):
        slot = c % 2
        for h in pending:
            h.wait()
        if c + 1 < _CHUNKS:
            pending = start_chunk(c + 1, (c + 1) % 2)
        acc = lax.fori_loop(
            0, _R, make_row_body(sbufs[slot], pbufs[slot], nbufs[slot]), acc)

    outv[...] = acc * (1.0 / (_B * _P))
    pltpu.sync_copy(outv, out_hbm.at[wid])


@jax.jit
def kernel(saliency_scores, pos_indices, neg_indices):
    mesh = plsc.VectorSubcoreMesh(core_axis_name="c", subcore_axis_name="s")
    run = functools.partial(
        pl.kernel,
        out_type=jax.ShapeDtypeStruct((_NW, _L), jnp.float32),
        mesh=mesh,
        compiler_params=pltpu.CompilerParams(needs_layout_passes=False),
        scratch_types=[
            pltpu.VMEM((_R, _N), jnp.float32),
            pltpu.VMEM((_R, _N), jnp.float32),
            pltpu.VMEM((_R, _P), jnp.int32),
            pltpu.VMEM((_R, _P), jnp.int32),
            pltpu.VMEM((_R, _P), jnp.int32),
            pltpu.VMEM((_R, _P), jnp.int32),
            pltpu.VMEM((_L,), jnp.float32),
            pltpu.SemaphoreType.DMA,
            pltpu.SemaphoreType.DMA,
        ],
    )(_loss_kernel)
    partials = run(
        saliency_scores,
        pos_indices.astype(jnp.int32),
        neg_indices.astype(jnp.int32),
    )
    return jnp.sum(partials)
